# Initial kernel scaffold; baseline (speedup 1.0000x reference)
import jax
import jax.numpy as jnp
from jax.experimental import pallas as pl


def kernel(x1, x2, emb, W1, b1, W2, b2):
    raise NotImplementedError("write your pallas kernel here")


# speedup vs baseline: 16.4185x; 16.4185x over previous
import functools

import jax
import jax.numpy as jnp
from jax import lax
from jax.experimental import pallas as pl
from jax.experimental.pallas import tpu as pltpu
from jax.experimental.pallas import tpu_sc as plsc

BATCH = 16384
LIN_IN = 13
N_CATS = 26
EMB_DIM = 16
HIDDEN = 256
OUT = 6

NUM_IDX = BATCH * N_CATS
NUM_WORKERS = 32
ROWS_PER_W = NUM_IDX // NUM_WORKERS
N_CHUNKS = 4
CHUNK = ROWS_PER_W // N_CHUNKS


def _gather_body(table_hbm, idx_hbm, out_hbm, idx_v, rows_v, sem):
    wid = lax.axis_index("s") * 2 + lax.axis_index("c")
    base = wid * ROWS_PER_W
    pltpu.sync_copy(idx_hbm.at[pl.ds(base, ROWS_PER_W)], idx_v)
    for ci in range(N_CHUNKS):
        off = ci * CHUNK
        pltpu.async_copy(
            table_hbm.at[idx_v.at[pl.ds(off, CHUNK)]], rows_v, sem
        ).wait()
        pltpu.sync_copy(rows_v, out_hbm.at[pl.ds(base + off, CHUNK)])


@functools.cache
def _make_gather():
    return pl.kernel(
        _gather_body,
        out_type=jax.ShapeDtypeStruct((NUM_IDX, EMB_DIM), jnp.float32),
        scratch_types=[
            pltpu.VMEM((ROWS_PER_W,), jnp.int32),
            pltpu.VMEM((CHUNK, EMB_DIM), jnp.float32),
            pltpu.SemaphoreType.DMA,
        ],
        mesh=plsc.VectorSubcoreMesh(core_axis_name="c", subcore_axis_name="s"),
        compiler_params=pltpu.CompilerParams(use_tc_tiling_on_sc=False),
    )


BM = 1024


def _mlp_body(x1_ref, e_ref, w1a_ref, w1b_ref, b1_ref, w2_ref, b2_ref, o_ref):
    acc = jnp.dot(x1_ref[...], w1a_ref[...], preferred_element_type=jnp.float32)
    acc += jnp.dot(e_ref[...], w1b_ref[...], preferred_element_type=jnp.float32)
    h = jnp.tanh(acc + b1_ref[...])
    o_ref[...] = (
        jnp.dot(h, w2_ref[...], preferred_element_type=jnp.float32) + b2_ref[...]
    )


@functools.partial(jax.jit, static_argnames=())
def _mlp(x1, e, w1a, w1b, b1, w2, b2):
    grid = (BATCH // BM,)
    return pl.pallas_call(
        _mlp_body,
        grid=grid,
        in_specs=[
            pl.BlockSpec((BM, LIN_IN), lambda i: (i, 0)),
            pl.BlockSpec((BM, N_CATS * EMB_DIM), lambda i: (i, 0)),
            pl.BlockSpec((LIN_IN, HIDDEN), lambda i: (0, 0)),
            pl.BlockSpec((N_CATS * EMB_DIM, HIDDEN), lambda i: (0, 0)),
            pl.BlockSpec((1, HIDDEN), lambda i: (0, 0)),
            pl.BlockSpec((HIDDEN, OUT), lambda i: (0, 0)),
            pl.BlockSpec((1, OUT), lambda i: (0, 0)),
        ],
        out_specs=pl.BlockSpec((BM, OUT), lambda i: (i, 0)),
        out_shape=jax.ShapeDtypeStruct((BATCH, OUT), jnp.float32),
    )(x1, e, w1a, w1b, b1, w2, b2)


def kernel(x1, x2, emb, W1, b1, W2, b2):
    idx = x2.astype(jnp.int32).reshape(-1)
    e = _make_gather()(emb, idx)
    e = e.reshape(BATCH, N_CATS * EMB_DIM)
    return _mlp(
        x1,
        e,
        W1[:LIN_IN],
        W1[LIN_IN:],
        b1.reshape(1, HIDDEN),
        W2,
        b2.reshape(1, OUT),
    )
